# baseline (speedup 1.0000x reference)
import functools

import jax
import jax.numpy as jnp
from jax import lax
from jax.experimental import pallas as pl
from jax.experimental.pallas import tpu as pltpu
from jax.experimental.pallas import tpu_sc as plsc

EMB = 128
R0, R1, R2 = 6, 7, 3
NT = R0 * R1 * R2
NC, NS = 2, 16
NW = NC * NS
CHUNK = 128
LANES = 16


def _fuse_tables_kernel(w0_ref, w1_ref, w2_ref, t_ref):
    r = lax.broadcasted_iota(jnp.int32, (NT, 1), 0)
    a = r // (R1 * R2)
    b = (r // R2) % R1
    c = r % R2
    oh0 = (a == lax.broadcasted_iota(jnp.int32, (NT, R0), 1)).astype(jnp.float32)
    oh1 = (b == lax.broadcasted_iota(jnp.int32, (NT, R1), 1)).astype(jnp.float32)
    oh2 = (c == lax.broadcasted_iota(jnp.int32, (NT, R2), 1)).astype(jnp.float32)
    acc = jnp.dot(oh0, w0_ref[...], preferred_element_type=jnp.float32)
    acc = acc + jnp.dot(oh1, w1_ref[...], preferred_element_type=jnp.float32)
    acc = acc + jnp.dot(oh2, w2_ref[...], preferred_element_type=jnp.float32)
    t_ref[...] = acc


def _fuse_tables(W0, W1, W2):
    return pl.pallas_call(
        _fuse_tables_kernel,
        out_shape=jax.ShapeDtypeStruct((NT, EMB), jnp.float32),
    )(W0, W1, W2)


NBUF = 6


def _sc_lookup_body(num_chunks, num_edges, ea_hbm, t_hbm, out_hbm, t_sh,
                    ea_v0, ea_v1, ea_v2, ea_v3, ea_v4, ea_v5,
                    idx_v0, idx_v1, idx_v2, idx_v3, idx_v4, idx_v5,
                    rows_v0, rows_v1, rows_v2, rows_v3, rows_v4, rows_v5,
                    in_sem0, in_sem1, in_sem2, in_sem3, in_sem4, in_sem5,
                    gat_sem0, gat_sem1, gat_sem2, gat_sem3, gat_sem4, gat_sem5,
                    out_sem0, out_sem1, out_sem2, out_sem3, out_sem4,
                    out_sem5):
    wid = lax.axis_index("s") * NC + lax.axis_index("c")

    @pl.when(lax.axis_index("s") == 0)
    def _():
        pltpu.sync_copy(t_hbm, t_sh)

    plsc.subcore_barrier()
    base_n = num_chunks // NW
    rem = num_chunks - base_n * NW
    n = base_n + jnp.where(wid < rem, 1, 0).astype(jnp.int32)

    ea_bufs = (ea_v0, ea_v1, ea_v2, ea_v3, ea_v4, ea_v5)
    idx_bufs = (idx_v0, idx_v1, idx_v2, idx_v3, idx_v4, idx_v5)
    rows_bufs = (rows_v0, rows_v1, rows_v2, rows_v3, rows_v4, rows_v5)
    in_sems = (in_sem0, in_sem1, in_sem2, in_sem3, in_sem4, in_sem5)
    gat_sems = (gat_sem0, gat_sem1, gat_sem2, gat_sem3, gat_sem4, gat_sem5)
    out_sems = (out_sem0, out_sem1, out_sem2, out_sem3, out_sem4, out_sem5)

    def start_in(i, b):
        chunk = wid + i * NW
        pltpu.async_copy(
            ea_hbm.at[pl.ds(pl.multiple_of(chunk * (3 * CHUNK), 8),
                            3 * CHUNK)],
            ea_bufs[b], in_sems[b])

    def wait_in(b):
        pltpu.make_async_copy(ea_hbm.at[pl.ds(0, CHUNK * 3)],
                              ea_bufs[b], in_sems[b]).wait()

    def start_out(i, b):
        ebase = (wid + i * NW) * CHUNK
        pltpu.async_copy(rows_bufs[b], out_hbm.at[pl.ds(ebase, CHUNK)],
                         out_sems[b])

    def wait_out(b):
        pltpu.make_async_copy(rows_bufs[b], out_hbm.at[pl.ds(0, CHUNK)],
                              out_sems[b]).wait()

    def start_gather(b):
        pltpu.async_copy(t_sh.at[idx_bufs[b]], rows_bufs[b], gat_sems[b])

    def wait_gather(b):
        pltpu.make_async_copy(t_sh.at[idx_bufs[b]], rows_bufs[b],
                              gat_sems[b]).wait()

    @pl.when(n > 0)
    def _():
        start_in(0, 0)

    @pl.when(n > 1)
    def _():
        start_in(1, 1)

    def step(i, b):
        bp = (b - 2) % NBUF

        @pl.when((i >= 2) & (i <= n + 1))
        def _():
            wait_gather(bp)
            start_out(i - 2, bp)

        @pl.when(i < n)
        def _():
            wait_in(b)
            for k in range(CHUNK // LANES):
                i0 = ea_bufs[b][pl.ds(k * LANES, LANES)]
                i1 = ea_bufs[b][pl.ds(CHUNK + k * LANES, LANES)]
                i2 = ea_bufs[b][pl.ds(2 * CHUNK + k * LANES, LANES)]
                i0 = jnp.clip(i0, 0, R0 - 1)
                i1 = jnp.clip(i1, 0, R1 - 1)
                i2 = jnp.clip(i2, 0, R2 - 1)
                idx_bufs[b][pl.ds(k * LANES, LANES)] = \
                    (i0 * R1 + i1) * R2 + i2

            @pl.when(i >= NBUF)
            def _():
                wait_out(b)

            start_gather(b)

            @pl.when(i + 2 < n)
            def _():
                start_in(i + 2, (b + 2) % NBUF)

    def gbody(g, carry):
        for b in range(NBUF):
            i = g * NBUF + b
            step(i, b)
        return carry

    lax.fori_loop(0, (n + 1) // NBUF + 1, gbody, 0)

    for k in range(NBUF):
        @pl.when(n > k)
        def _():
            wait_out(k)


def _sc_lookup(ea_flat, T, E):
    num_chunks = E // CHUNK
    mesh = plsc.VectorSubcoreMesh(core_axis_name="c", subcore_axis_name="s")
    return pl.kernel(
        functools.partial(_sc_lookup_body, num_chunks, E),
        out_type=jax.ShapeDtypeStruct((E, EMB), jnp.float32),
        mesh=mesh,
        compiler_params=pltpu.CompilerParams(needs_layout_passes=False),
        scratch_types=(
            [pltpu.VMEM_SHARED((NT, EMB), jnp.float32)]
            + [pltpu.VMEM((CHUNK * 3,), jnp.int32)] * NBUF
            + [pltpu.VMEM((CHUNK,), jnp.int32)] * NBUF
            + [pltpu.VMEM((CHUNK, EMB), jnp.float32)] * NBUF
            + [pltpu.SemaphoreType.DMA] * (3 * NBUF)
        ),
    )(ea_flat, T)


def kernel(edge_attr, W0, W1, W2):
    E = edge_attr.shape[0]
    T = _fuse_tables(W0, W1, W2)
    ea_t = edge_attr.reshape(-1, CHUNK, 3).transpose(0, 2, 1).reshape(-1)
    return _sc_lookup(ea_t, T, E)


# speedup vs baseline: 1.0239x; 1.0239x over previous
import functools

import jax
import jax.numpy as jnp
from jax import lax
from jax.experimental import pallas as pl
from jax.experimental.pallas import tpu as pltpu
from jax.experimental.pallas import tpu_sc as plsc

EMB = 128
R0, R1, R2 = 6, 7, 3
NT = R0 * R1 * R2
NC, NS = 2, 16
NW = NC * NS
CHUNK = 128
LANES = 16


def _fuse_tables_kernel(w0_ref, w1_ref, w2_ref, t_ref):
    r = lax.broadcasted_iota(jnp.int32, (NT, 1), 0)
    a = r // (R1 * R2)
    b = (r // R2) % R1
    c = r % R2
    oh0 = (a == lax.broadcasted_iota(jnp.int32, (NT, R0), 1)).astype(jnp.float32)
    oh1 = (b == lax.broadcasted_iota(jnp.int32, (NT, R1), 1)).astype(jnp.float32)
    oh2 = (c == lax.broadcasted_iota(jnp.int32, (NT, R2), 1)).astype(jnp.float32)
    acc = jnp.dot(oh0, w0_ref[...], preferred_element_type=jnp.float32)
    acc = acc + jnp.dot(oh1, w1_ref[...], preferred_element_type=jnp.float32)
    acc = acc + jnp.dot(oh2, w2_ref[...], preferred_element_type=jnp.float32)
    t_ref[...] = acc


def _fuse_tables(W0, W1, W2):
    return pl.pallas_call(
        _fuse_tables_kernel,
        out_shape=jax.ShapeDtypeStruct((NT, EMB), jnp.float32),
    )(W0, W1, W2)


NBUF = 7


def _sc_lookup_body(num_chunks, num_edges, ea_hbm, t_hbm, out_hbm, t_sh,
                    ea_v0, ea_v1, ea_v2, ea_v3, ea_v4, ea_v5, ea_v6,
                    idx_v0, idx_v1, idx_v2, idx_v3, idx_v4, idx_v5, idx_v6,
                    rows_v0, rows_v1, rows_v2, rows_v3, rows_v4, rows_v5,
                    rows_v6,
                    in_sem0, in_sem1, in_sem2, in_sem3, in_sem4, in_sem5,
                    in_sem6,
                    gat_sem0, gat_sem1, gat_sem2, gat_sem3, gat_sem4, gat_sem5,
                    gat_sem6,
                    out_sem0, out_sem1, out_sem2, out_sem3, out_sem4,
                    out_sem5, out_sem6):
    wid = lax.axis_index("s") * NC + lax.axis_index("c")

    @pl.when(lax.axis_index("s") == 0)
    def _():
        pltpu.sync_copy(t_hbm, t_sh)

    plsc.subcore_barrier()
    base_n = num_chunks // NW
    rem = num_chunks - base_n * NW
    n = base_n + jnp.where(wid < rem, 1, 0).astype(jnp.int32)

    ea_bufs = (ea_v0, ea_v1, ea_v2, ea_v3, ea_v4, ea_v5, ea_v6)
    idx_bufs = (idx_v0, idx_v1, idx_v2, idx_v3, idx_v4, idx_v5, idx_v6)
    rows_bufs = (rows_v0, rows_v1, rows_v2, rows_v3, rows_v4, rows_v5, rows_v6)
    in_sems = (in_sem0, in_sem1, in_sem2, in_sem3, in_sem4, in_sem5, in_sem6)
    gat_sems = (gat_sem0, gat_sem1, gat_sem2, gat_sem3, gat_sem4, gat_sem5, gat_sem6)
    out_sems = (out_sem0, out_sem1, out_sem2, out_sem3, out_sem4, out_sem5, out_sem6)

    def start_in(i, b):
        ebase = (wid + i * NW) * CHUNK
        for k in range(3):
            pltpu.async_copy(
                ea_hbm.at[pl.ds(pl.multiple_of(k * num_edges + ebase, 8),
                                CHUNK)],
                ea_bufs[b].at[pl.ds(k * CHUNK, CHUNK)], in_sems[b])

    def wait_in(b):
        pltpu.make_async_copy(ea_hbm.at[pl.ds(0, CHUNK * 3)],
                              ea_bufs[b], in_sems[b]).wait()

    def start_out(i, b):
        ebase = (wid + i * NW) * CHUNK
        pltpu.async_copy(rows_bufs[b], out_hbm.at[pl.ds(ebase, CHUNK)],
                         out_sems[b])

    def wait_out(b):
        pltpu.make_async_copy(rows_bufs[b], out_hbm.at[pl.ds(0, CHUNK)],
                              out_sems[b]).wait()

    def start_gather(b):
        pltpu.async_copy(t_sh.at[idx_bufs[b]], rows_bufs[b], gat_sems[b])

    def wait_gather(b):
        pltpu.make_async_copy(t_sh.at[idx_bufs[b]], rows_bufs[b],
                              gat_sems[b]).wait()

    @pl.when(n > 0)
    def _():
        start_in(0, 0)

    @pl.when(n > 1)
    def _():
        start_in(1, 1)

    def step(i, b):
        bp = (b - 3) % NBUF

        @pl.when((i >= 3) & (i <= n + 2))
        def _():
            wait_gather(bp)
            start_out(i - 3, bp)

        @pl.when(i < n)
        def _():
            wait_in(b)
            for k in range(CHUNK // LANES):
                i0 = ea_bufs[b][pl.ds(k * LANES, LANES)]
                i1 = ea_bufs[b][pl.ds(CHUNK + k * LANES, LANES)]
                i2 = ea_bufs[b][pl.ds(2 * CHUNK + k * LANES, LANES)]
                i0 = jnp.clip(i0, 0, R0 - 1)
                i1 = jnp.clip(i1, 0, R1 - 1)
                i2 = jnp.clip(i2, 0, R2 - 1)
                idx_bufs[b][pl.ds(k * LANES, LANES)] = \
                    (i0 * R1 + i1) * R2 + i2

            @pl.when(i >= NBUF)
            def _():
                wait_out(b)

            start_gather(b)

            @pl.when(i + 2 < n)
            def _():
                start_in(i + 2, (b + 2) % NBUF)

    def gbody(g, carry):
        for b in range(NBUF):
            i = g * NBUF + b
            step(i, b)
        return carry

    lax.fori_loop(0, (n + 2) // NBUF + 1, gbody, 0)

    for k in range(NBUF):
        @pl.when(n > k)
        def _():
            wait_out(k)


def _sc_lookup(ea_flat, T, E):
    num_chunks = E // CHUNK
    mesh = plsc.VectorSubcoreMesh(core_axis_name="c", subcore_axis_name="s")
    return pl.kernel(
        functools.partial(_sc_lookup_body, num_chunks, E),
        out_type=jax.ShapeDtypeStruct((E, EMB), jnp.float32),
        mesh=mesh,
        compiler_params=pltpu.CompilerParams(needs_layout_passes=False),
        scratch_types=(
            [pltpu.VMEM_SHARED((NT, EMB), jnp.float32)]
            + [pltpu.VMEM((CHUNK * 3,), jnp.int32)] * NBUF
            + [pltpu.VMEM((CHUNK,), jnp.int32)] * NBUF
            + [pltpu.VMEM((CHUNK, EMB), jnp.float32)] * NBUF
            + [pltpu.SemaphoreType.DMA] * (3 * NBUF)
        ),
    )(ea_flat, T)


def kernel(edge_attr, W0, W1, W2):
    E = edge_attr.shape[0]
    T = _fuse_tables(W0, W1, W2)
    ea_t = edge_attr.T.reshape(-1)
    return _sc_lookup(ea_t, T, E)
